# baseline (speedup 1.0000x reference)
import functools

import jax
import jax.numpy as jnp
from jax import lax
from jax.experimental import pallas as pl
from jax.experimental.pallas import tpu as pltpu
from jax.experimental.pallas import tpu_sc as plsc

F = 26
V = 100001
D = 50
DP = 56
B = 16384
R = B * F

_info = plsc.get_sparse_core_info()
NC, NS, L = _info.num_cores, _info.num_subcores, _info.num_lanes
NW = NC * NS
RW = R // NW
C = 512
NCHUNK = RW // C
KB = 128
KG = C // KB

_mesh = plsc.VectorSubcoreMesh(core_axis_name="c", subcore_axis_name="s")


@functools.partial(
    pl.kernel,
    mesh=_mesh,
    out_type=jax.ShapeDtypeStruct((R, DP), jnp.float32),
    compiler_params=pltpu.CompilerParams(use_tc_tiling_on_sc=False),
    scratch_types=[
        pltpu.VMEM((KB,), jnp.int32),
        pltpu.VMEM((KB,), jnp.int32),
        pltpu.VMEM((KB,), jnp.int32),
        pltpu.VMEM((KB,), jnp.int32),
        pltpu.VMEM((C, DP), jnp.float32),
        pltpu.SemaphoreType.DMA,
    ],
)
def _gather_kernel(gidx_hbm, tab_hbm, out_hbm, g0, g1, g2, g3, rows_v, sem):
    gidx = [g0, g1, g2, g3]
    wid = lax.axis_index("s") * NC + lax.axis_index("c")
    base = wid * RW

    def chunk(g, carry):
        rowbase = base + g * C
        for k in range(KG):
            pltpu.sync_copy(gidx_hbm.at[pl.ds(rowbase + k * KB, KB)], gidx[k])
        copies = [
            pltpu.async_copy(
                tab_hbm.at[gidx[k]],
                rows_v.at[pl.ds(k * KB, KB)],
                sem,
            )
            for k in range(KG)
        ]
        for cp in copies:
            cp.wait()
        pltpu.sync_copy(rows_v, out_hbm.at[pl.ds(rowbase, C)])
        return carry

    lax.fori_loop(0, NCHUNK, chunk, 0)


_offsets = jnp.arange(F, dtype=jnp.int32) * V


def kernel(x, tables):
    gidx = (x + _offsets[None, :]).reshape(R)
    tab_pad = jnp.pad(tables, ((0, 0), (0, 0), (0, DP - D))).reshape(F * V, DP)
    out = _gather_kernel(gidx, tab_pad)
    return out[:, :D].reshape(B, F * D)


# speedup vs baseline: 2.1331x; 2.1331x over previous
import functools

import jax
import jax.numpy as jnp
from jax import lax
from jax.experimental import pallas as pl
from jax.experimental.pallas import tpu as pltpu
from jax.experimental.pallas import tpu_sc as plsc

F = 26
V = 100001
D = 50
DP = 56
B = 16384
R = B * F

_info = plsc.get_sparse_core_info()
NC, NS, L = _info.num_cores, _info.num_subcores, _info.num_lanes
NW = NC * NS
RW = R // NW
C = 512
NCHUNK = RW // C
KB = 128
KG = C // KB

_mesh = plsc.VectorSubcoreMesh(core_axis_name="c", subcore_axis_name="s")


@functools.partial(
    pl.kernel,
    mesh=_mesh,
    out_type=jax.ShapeDtypeStruct((R, DP), jnp.float32),
    compiler_params=pltpu.CompilerParams(use_tc_tiling_on_sc=False),
    scratch_types=[
        pltpu.VMEM((KB,), jnp.int32),
        pltpu.VMEM((KB,), jnp.int32),
        pltpu.VMEM((KB,), jnp.int32),
        pltpu.VMEM((KB,), jnp.int32),
        pltpu.VMEM((C, DP), jnp.float32),
        pltpu.SemaphoreType.DMA,
    ],
)
def _gather_kernel(gidx_hbm, tab_hbm, out_hbm, g0, g1, g2, g3, rows_v, sem):
    gidx = [g0, g1, g2, g3]
    wid = lax.axis_index("s") * NC + lax.axis_index("c")
    base = wid * RW

    def chunk(g, carry):
        rowbase = base + g * C
        for k in range(KG):
            pltpu.sync_copy(gidx_hbm.at[pl.ds(rowbase + k * KB, KB)], gidx[k])
        copies = [
            pltpu.async_copy(
                tab_hbm.at[gidx[k]],
                rows_v.at[pl.ds(k * KB, KB)],
                sem,
            )
            for k in range(KG)
        ]
        for cp in copies:
            cp.wait()
        pltpu.sync_copy(rows_v, out_hbm.at[pl.ds(rowbase, C)])
        return carry

    lax.fori_loop(0, NCHUNK, chunk, 0)


_offsets = jnp.arange(F, dtype=jnp.int32) * V


def kernel(x, tables):
    gidx = (x + _offsets[None, :]).reshape(R)
    tab_pad = jnp.concatenate(
        [tables.reshape(F * V, D),
         jnp.zeros((F * V, DP - D), jnp.float32)], axis=1)
    out = _gather_kernel(gidx, tab_pad)
    return out[:, :D].reshape(B, F * D)
